# baseline (speedup 1.0000x reference)
import functools

import jax
import jax.numpy as jnp
from jax import lax
from jax.experimental import pallas as pl
from jax.experimental.pallas import tpu as pltpu
from jax.experimental.pallas import tpu_sc as plsc

NC = 2
NS = 16
L = 16
K = 128

_DN = lax.GatherDimensionNumbers(
    offset_dims=(), collapsed_slice_dims=(0,), start_index_map=(0,))


def _lane_splat(vec, j):
    return lax.gather(vec, jnp.full((L, 1), j, jnp.int32), _DN, (1,),
                      mode=lax.GatherScatterMode.PROMISE_IN_BOUNDS)


def _sc_edge_kernel(x, epack, vals3d):
    n, d = x.shape
    nsub = epack.shape[0]
    nw = NC * NS
    base_k = nsub // nw
    extra = nsub - base_k * nw
    npair = base_k // 2
    assert base_k % 2 == 0
    RC = 80
    nrc = n // RC
    assert nrc * RC == n

    mesh = plsc.VectorSubcoreMesh(core_axis_name="c", subcore_axis_name="s")

    @functools.partial(
        pl.kernel,
        out_type=jax.ShapeDtypeStruct((NC, n, d), jnp.float32),
        mesh=mesh,
        scratch_types=[
            pltpu.VMEM((2, K), jnp.int32),
            pltpu.VMEM((K // L, L), jnp.float32),
            pltpu.VMEM((K,), jnp.int32),
            pltpu.VMEM((K, d), jnp.float32),
            pltpu.VMEM((2, K), jnp.int32),
            pltpu.VMEM((K // L, L), jnp.float32),
            pltpu.VMEM((K,), jnp.int32),
            pltpu.VMEM((K, d), jnp.float32),
            pltpu.VMEM((8, d), jnp.float32),
            pltpu.VMEM_SHARED((n, d), jnp.float32),
            pltpu.SemaphoreType.DMA,
            pltpu.SemaphoreType.DMA,
            pltpu.SemaphoreType.DMA,
            pltpu.SemaphoreType.DMA,
            pltpu.SemaphoreType.DMA,
            pltpu.SemaphoreType.DMA,
            pltpu.SemaphoreType.DMA,
        ],
    )
    def sc_kernel(x_hbm, ep_hbm, vals_hbm, out_hbm,
                  ebufA, valsA, didxA, rowsA, ebufB, valsB, didxB, rowsB,
                  zrow_v, acc_sh, gsemA, gsemB, isemA, isemB,
                  ssemA, ssemB, zsem):
        cid = lax.axis_index("c")
        sid = lax.axis_index("s")
        wid = cid * NS + sid
        sets = ((ebufA, valsA, didxA, rowsA, gsemA, isemA, ssemA),
                (ebufB, valsB, didxB, rowsB, gsemB, isemB, ssemB))

        def chunk_of(k):
            return wid + nw * k

        def issue_idx(k, st):
            ebuf, vv, _, _, _, isem, _ = st
            s = chunk_of(k)
            pltpu.async_copy(ep_hbm.at[s], ebuf, isem)
            pltpu.async_copy(vals_hbm.at[s], vv, isem)

        def wait_idx(k, st):
            ebuf, vv, _, _, _, isem, _ = st
            s = chunk_of(k)
            pltpu.make_async_copy(ep_hbm.at[s], ebuf, isem).wait()
            pltpu.make_async_copy(vals_hbm.at[s], vv, isem).wait()

        def issue_gather(st):
            ebuf, _, _, rows, gsem, _, _ = st
            pltpu.async_copy(x_hbm.at[ebuf.at[0]], rows, gsem)

        def wait_gather(st):
            ebuf, _, _, rows, gsem, _, _ = st
            pltpu.make_async_copy(x_hbm.at[ebuf.at[0]], rows, gsem).wait()

        def compute(st):
            _, vv, _, rows, _, _, _ = st

            def gbody(g, carry):
                vg = vv[g, :]
                for j in range(L):
                    vj = _lane_splat(vg, j)
                    gi = g * L + j
                    for c in range(d // L):
                        sl = slice(c * L, (c + 1) * L)
                        rows[gi, sl] = rows[gi, sl] * vj
                return carry
            lax.fori_loop(0, K // L, gbody, 0)

        def issue_scatter(st):
            ebuf, _, didx, rows, _, _, ssem = st
            for c in range(K // L):
                didx[c * L:(c + 1) * L] = ebuf[1, c * L:(c + 1) * L]
            pltpu.async_copy(rows, acc_sh.at[didx], ssem, add=True)

        def wait_scatter(st):
            _, _, didx, rows, _, _, ssem = st
            pltpu.make_async_copy(rows, acc_sh.at[didx], ssem).wait()

        issue_idx(0, sets[0])
        issue_idx(1, sets[1])

        for r in range(8):
            for c in range(d // L):
                zrow_v[r, c * L:(c + 1) * L] = jnp.zeros((L,), jnp.float32)

        def zero_issue(i, carry):
            rchunk = sid + i * NS

            @pl.when(rchunk < nrc)
            def _():
                for jj in range(RC // 8):
                    pltpu.async_copy(
                        zrow_v, acc_sh.at[pl.ds(rchunk * RC + jj * 8, 8)],
                        zsem)
            return carry
        lax.fori_loop(0, (nrc + NS - 1) // NS, zero_issue, 0)

        wait_idx(0, sets[0])
        issue_gather(sets[0])

        def zero_wait(i, carry):
            rchunk = sid + i * NS

            @pl.when(rchunk < nrc)
            def _():
                for jj in range(RC // 8):
                    pltpu.make_async_copy(
                        zrow_v, acc_sh.at[pl.ds(rchunk * RC + jj * 8, 8)],
                        zsem).wait()
            return carry
        lax.fori_loop(0, (nrc + NS - 1) // NS, zero_wait, 0)
        plsc.subcore_barrier()

        has_extra = wid < extra

        def stage(k, cur, nxt, guard_next, guard_nn, first=False):
            @pl.when(guard_next)
            def _():
                wait_idx(k + 1, nxt)
                if not first:
                    wait_scatter(nxt)
                issue_gather(nxt)
            wait_gather(cur)
            compute(cur)
            issue_scatter(cur)

            @pl.when(guard_nn)
            def _():
                issue_idx(k + 2, cur)

        def pair_body(i, carry):
            k0 = 2 * i
            last = i == npair - 1
            stage(k0, sets[0], sets[1],
                  jnp.bool_(True),
                  jnp.logical_or(~last, has_extra))
            stage(k0 + 1, sets[1], sets[0],
                  jnp.logical_or(~last, has_extra),
                  ~last)
            return carry

        stage(0, sets[0], sets[1], jnp.bool_(True),
              jnp.bool_(npair > 1) | has_extra, first=True)
        stage(1, sets[1], sets[0],
              jnp.bool_(npair > 1) | has_extra,
              jnp.bool_(npair > 1))
        lax.fori_loop(1, npair, pair_body, 0)

        @pl.when(has_extra)
        def _():
            wait_gather(sets[0])
            compute(sets[0])
            issue_scatter(sets[0])
        wait_scatter(sets[1])
        wait_scatter(sets[0])

        plsc.subcore_barrier()

        def rb_issue(i, carry):
            rchunk = sid + i * NS

            @pl.when(rchunk < nrc)
            def _():
                r = rchunk * RC
                pltpu.async_copy(acc_sh.at[pl.ds(r, RC)],
                                 out_hbm.at[cid, pl.ds(r, RC)], zsem)
            return carry
        lax.fori_loop(0, (nrc + NS - 1) // NS, rb_issue, 0)

        def rb_wait(i, carry):
            rchunk = sid + i * NS

            @pl.when(rchunk < nrc)
            def _():
                r = rchunk * RC
                pltpu.make_async_copy(acc_sh.at[pl.ds(r, RC)],
                                      out_hbm.at[cid, pl.ds(r, RC)],
                                      zsem).wait()
            return carry
        lax.fori_loop(0, (nrc + NS - 1) // NS, rb_wait, 0)

    return sc_kernel(x, epack, vals3d)


def _dense_body(acc_ref, x_ref, w_ref, b_ref, o_ref):
    a = acc_ref[0] + acc_ref[1] + x_ref[...]
    s = jnp.dot(a, w_ref[...], preferred_element_type=jnp.float32) + b_ref[...]
    alpha = 1.6732632423543772
    scale = 1.0507009873554805
    o_ref[...] = scale * jnp.where(s > 0, s, alpha * (jnp.exp(s) - 1.0))


def _tc_dense(acc, x, w, b):
    n, d = x.shape
    h = w.shape[1]
    bm = 400
    grid = (n // bm,)
    return pl.pallas_call(
        _dense_body,
        grid=grid,
        in_specs=[
            pl.BlockSpec((NC, bm, d), lambda i: (0, i, 0)),
            pl.BlockSpec((bm, d), lambda i: (i, 0)),
            pl.BlockSpec((d, h), lambda i: (0, 0)),
            pl.BlockSpec((1, h), lambda i: (0, 0)),
        ],
        out_specs=pl.BlockSpec((bm, h), lambda i: (i, 0)),
        out_shape=jax.ShapeDtypeStruct((n, h), jnp.float32),
    )(acc, x, w, b)


@jax.jit
def kernel(x, edge_index, edge_vals, W0, b0):
    e = edge_vals.shape[0]
    nsub = e // K
    src = edge_index[0].astype(jnp.int32).reshape(nsub, K)
    dst = edge_index[1].astype(jnp.int32).reshape(nsub, K)
    epack = jnp.stack([src, dst], axis=1)
    vals3d = edge_vals.astype(jnp.float32).reshape(nsub, K // L, L)
    acc = _sc_edge_kernel(x, epack, vals3d)
    return _tc_dense(acc, x, W0, b0.reshape(1, -1))


# speedup vs baseline: 1.0454x; 1.0454x over previous
import functools

import jax
import jax.numpy as jnp
from jax import lax
from jax.experimental import pallas as pl
from jax.experimental.pallas import tpu as pltpu
from jax.experimental.pallas import tpu_sc as plsc

NC = 2
NS = 16
L = 16
K = 128

_DN = lax.GatherDimensionNumbers(
    offset_dims=(), collapsed_slice_dims=(0,), start_index_map=(0,))


def _lane_splat(vec, j):
    return lax.gather(vec, jnp.full((L, 1), j, jnp.int32), _DN, (1,),
                      mode=lax.GatherScatterMode.PROMISE_IN_BOUNDS)


def _sc_edge_kernel(x, epack, vals3d):
    n, d = x.shape
    nsub = epack.shape[0]
    nw = NC * NS
    base_k = nsub // nw
    extra = nsub - base_k * nw
    npair = base_k // 2
    assert base_k % 2 == 0
    RC = 80
    nrc = n // RC
    assert nrc * RC == n

    mesh = plsc.VectorSubcoreMesh(core_axis_name="c", subcore_axis_name="s")

    @functools.partial(
        pl.kernel,
        out_type=jax.ShapeDtypeStruct((NC, n, d), jnp.float32),
        mesh=mesh,
        scratch_types=[
            pltpu.VMEM((2, K), jnp.int32),
            pltpu.VMEM((K // L, L), jnp.float32),
            pltpu.VMEM((K,), jnp.int32),
            pltpu.VMEM((K, d), jnp.float32),
            pltpu.VMEM((2, K), jnp.int32),
            pltpu.VMEM((K // L, L), jnp.float32),
            pltpu.VMEM((K,), jnp.int32),
            pltpu.VMEM((K, d), jnp.float32),
            pltpu.VMEM((8, d), jnp.float32),
            pltpu.VMEM_SHARED((n, d), jnp.float32),
            pltpu.SemaphoreType.DMA,
            pltpu.SemaphoreType.DMA,
            pltpu.SemaphoreType.DMA,
            pltpu.SemaphoreType.DMA,
            pltpu.SemaphoreType.DMA,
            pltpu.SemaphoreType.DMA,
            pltpu.SemaphoreType.DMA,
        ],
    )
    def sc_kernel(x_hbm, ep_hbm, vals_hbm, out_hbm,
                  ebufA, valsA, didxA, rowsA, ebufB, valsB, didxB, rowsB,
                  zrow_v, acc_sh, gsemA, gsemB, isemA, isemB,
                  ssemA, ssemB, zsem):
        cid = lax.axis_index("c")
        sid = lax.axis_index("s")
        wid = cid * NS + sid
        sets = ((ebufA, valsA, didxA, rowsA, gsemA, isemA, ssemA),
                (ebufB, valsB, didxB, rowsB, gsemB, isemB, ssemB))

        def chunk_of(k):
            return wid + nw * k

        def issue_idx(k, st):
            ebuf, vv, _, _, _, isem, _ = st
            s = chunk_of(k)
            pltpu.async_copy(ep_hbm.at[s], ebuf, isem)
            pltpu.async_copy(vals_hbm.at[s], vv, isem)

        def wait_idx(k, st):
            ebuf, vv, _, _, _, isem, _ = st
            s = chunk_of(k)
            pltpu.make_async_copy(ep_hbm.at[s], ebuf, isem).wait()
            pltpu.make_async_copy(vals_hbm.at[s], vv, isem).wait()

        def issue_gather(st):
            ebuf, _, _, rows, gsem, _, _ = st
            pltpu.async_copy(x_hbm.at[ebuf.at[0]], rows, gsem)

        def wait_gather(st):
            ebuf, _, _, rows, gsem, _, _ = st
            pltpu.make_async_copy(x_hbm.at[ebuf.at[0]], rows, gsem).wait()

        def compute(st):
            _, vv, _, rows, _, _, _ = st

            def gbody(g, carry):
                vg = vv[g, :]
                for j in range(L):
                    vj = _lane_splat(vg, j)
                    gi = g * L + j
                    for c in range(d // L):
                        sl = slice(c * L, (c + 1) * L)
                        rows[gi, sl] = rows[gi, sl] * vj
                return carry
            lax.fori_loop(0, K // L, gbody, 0)

        def issue_scatter(st):
            ebuf, _, didx, rows, _, _, ssem = st
            for c in range(K // L):
                didx[c * L:(c + 1) * L] = ebuf[1, c * L:(c + 1) * L]
            pltpu.async_copy(rows, acc_sh.at[didx], ssem, add=True)

        def wait_scatter(st):
            _, _, didx, rows, _, _, ssem = st
            pltpu.make_async_copy(rows, acc_sh.at[didx], ssem).wait()

        issue_idx(0, sets[0])
        issue_idx(1, sets[1])

        for r in range(8):
            for c in range(d // L):
                zrow_v[r, c * L:(c + 1) * L] = jnp.zeros((L,), jnp.float32)

        def init_issue(i, carry):
            rchunk = sid + i * NS

            @pl.when(rchunk < nrc)
            def _():
                @pl.when(cid == 0)
                def _():
                    pltpu.async_copy(
                        x_hbm.at[pl.ds(rchunk * RC, RC)],
                        acc_sh.at[pl.ds(rchunk * RC, RC)], zsem)

                @pl.when(cid != 0)
                def _():
                    for jj in range(RC // 8):
                        pltpu.async_copy(
                            zrow_v, acc_sh.at[pl.ds(rchunk * RC + jj * 8, 8)],
                            zsem)
            return carry
        lax.fori_loop(0, (nrc + NS - 1) // NS, init_issue, 0)

        wait_idx(0, sets[0])
        issue_gather(sets[0])

        def init_wait(i, carry):
            rchunk = sid + i * NS

            @pl.when(rchunk < nrc)
            def _():
                @pl.when(cid == 0)
                def _():
                    pltpu.make_async_copy(
                        x_hbm.at[pl.ds(rchunk * RC, RC)],
                        acc_sh.at[pl.ds(rchunk * RC, RC)], zsem).wait()

                @pl.when(cid != 0)
                def _():
                    for jj in range(RC // 8):
                        pltpu.make_async_copy(
                            zrow_v, acc_sh.at[pl.ds(rchunk * RC + jj * 8, 8)],
                            zsem).wait()
            return carry
        lax.fori_loop(0, (nrc + NS - 1) // NS, init_wait, 0)
        plsc.subcore_barrier()

        has_extra = wid < extra

        def stage(k, cur, nxt, guard_next, guard_nn, first=False):
            @pl.when(guard_next)
            def _():
                wait_idx(k + 1, nxt)
                if not first:
                    wait_scatter(nxt)
                issue_gather(nxt)
            wait_gather(cur)
            compute(cur)
            issue_scatter(cur)

            @pl.when(guard_nn)
            def _():
                issue_idx(k + 2, cur)

        def pair_body(i, carry):
            k0 = 2 * i
            last = i == npair - 1
            stage(k0, sets[0], sets[1],
                  jnp.bool_(True),
                  jnp.logical_or(~last, has_extra))
            stage(k0 + 1, sets[1], sets[0],
                  jnp.logical_or(~last, has_extra),
                  ~last)
            return carry

        stage(0, sets[0], sets[1], jnp.bool_(True),
              jnp.bool_(npair > 1) | has_extra, first=True)
        stage(1, sets[1], sets[0],
              jnp.bool_(npair > 1) | has_extra,
              jnp.bool_(npair > 1))
        lax.fori_loop(1, npair, pair_body, 0)

        @pl.when(has_extra)
        def _():
            wait_gather(sets[0])
            compute(sets[0])
            issue_scatter(sets[0])
        wait_scatter(sets[1])
        wait_scatter(sets[0])

        plsc.subcore_barrier()

        def rb_issue(i, carry):
            rchunk = sid + i * NS

            @pl.when(rchunk < nrc)
            def _():
                r = rchunk * RC
                pltpu.async_copy(acc_sh.at[pl.ds(r, RC)],
                                 out_hbm.at[cid, pl.ds(r, RC)], zsem)
            return carry
        lax.fori_loop(0, (nrc + NS - 1) // NS, rb_issue, 0)

        def rb_wait(i, carry):
            rchunk = sid + i * NS

            @pl.when(rchunk < nrc)
            def _():
                r = rchunk * RC
                pltpu.make_async_copy(acc_sh.at[pl.ds(r, RC)],
                                      out_hbm.at[cid, pl.ds(r, RC)],
                                      zsem).wait()
            return carry
        lax.fori_loop(0, (nrc + NS - 1) // NS, rb_wait, 0)

    return sc_kernel(x, epack, vals3d)


def _dense_body(acc_ref, w_ref, b_ref, o_ref):
    a = acc_ref[0] + acc_ref[1]
    s = jnp.dot(a, w_ref[...], preferred_element_type=jnp.float32) + b_ref[...]
    alpha = 1.6732632423543772
    scale = 1.0507009873554805
    o_ref[...] = scale * jnp.where(s > 0, s, alpha * (jnp.exp(s) - 1.0))


def _tc_dense(acc, w, b):
    nc, n, d = acc.shape
    h = w.shape[1]
    bm = 2000
    grid = (n // bm,)
    return pl.pallas_call(
        _dense_body,
        grid=grid,
        in_specs=[
            pl.BlockSpec((NC, bm, d), lambda i: (0, i, 0)),
            pl.BlockSpec((d, h), lambda i: (0, 0)),
            pl.BlockSpec((1, h), lambda i: (0, 0)),
        ],
        out_specs=pl.BlockSpec((bm, h), lambda i: (i, 0)),
        out_shape=jax.ShapeDtypeStruct((n, h), jnp.float32),
    )(acc, w, b)


@jax.jit
def kernel(x, edge_index, edge_vals, W0, b0):
    e = edge_vals.shape[0]
    nsub = e // K
    src = edge_index[0].astype(jnp.int32).reshape(nsub, K)
    dst = edge_index[1].astype(jnp.int32).reshape(nsub, K)
    epack = jnp.stack([src, dst], axis=1)
    vals3d = edge_vals.astype(jnp.float32).reshape(nsub, K // L, L)
    acc = _sc_edge_kernel(x, epack, vals3d)
    return _tc_dense(acc, W0, b0.reshape(1, -1))


# speedup vs baseline: 1.0760x; 1.0292x over previous
import functools

import jax
import jax.numpy as jnp
from jax import lax
from jax.experimental import pallas as pl
from jax.experimental.pallas import tpu as pltpu
from jax.experimental.pallas import tpu_sc as plsc

NC = 2
NS = 16
L = 16
K = 128

_DN = lax.GatherDimensionNumbers(
    offset_dims=(), collapsed_slice_dims=(0,), start_index_map=(0,))


def _lane_splat(vec, j):
    return lax.gather(vec, jnp.full((L, 1), j, jnp.int32), _DN, (1,),
                      mode=lax.GatherScatterMode.PROMISE_IN_BOUNDS)


def _sc_edge_kernel(x, ei3, vals3d):
    n, d = x.shape
    nsub = ei3.shape[1]
    nw = NC * NS
    base_k = nsub // nw
    extra = nsub - base_k * nw
    npair = base_k // 2
    assert base_k % 2 == 0
    RC = 80
    nrc = n // RC
    assert nrc * RC == n

    mesh = plsc.VectorSubcoreMesh(core_axis_name="c", subcore_axis_name="s")

    @functools.partial(
        pl.kernel,
        out_type=jax.ShapeDtypeStruct((NC, n, d), jnp.float32),
        mesh=mesh,
        scratch_types=[
            pltpu.VMEM((K,), jnp.int32),
            pltpu.VMEM((K,), jnp.int32),
            pltpu.VMEM((K // L, L), jnp.float32),
            pltpu.VMEM((K,), jnp.int32),
            pltpu.VMEM((K, d), jnp.float32),
            pltpu.VMEM((K,), jnp.int32),
            pltpu.VMEM((K,), jnp.int32),
            pltpu.VMEM((K // L, L), jnp.float32),
            pltpu.VMEM((K,), jnp.int32),
            pltpu.VMEM((K, d), jnp.float32),
            pltpu.VMEM((8, d), jnp.float32),
            pltpu.VMEM_SHARED((n, d), jnp.float32),
            pltpu.SemaphoreType.DMA,
            pltpu.SemaphoreType.DMA,
            pltpu.SemaphoreType.DMA,
            pltpu.SemaphoreType.DMA,
            pltpu.SemaphoreType.DMA,
            pltpu.SemaphoreType.DMA,
            pltpu.SemaphoreType.DMA,
        ],
    )
    def sc_kernel(x_hbm, ei_hbm, vals_hbm, out_hbm,
                  sbufA, dbufA, valsA, didxA, rowsA,
                  sbufB, dbufB, valsB, didxB, rowsB,
                  zrow_v, acc_sh, gsemA, gsemB, isemA, isemB,
                  ssemA, ssemB, zsem):
        cid = lax.axis_index("c")
        sid = lax.axis_index("s")
        wid = cid * NS + sid
        sets = ((sbufA, dbufA, valsA, didxA, rowsA, gsemA, isemA, ssemA),
                (sbufB, dbufB, valsB, didxB, rowsB, gsemB, isemB, ssemB))

        def chunk_of(k):
            return wid + nw * k

        def issue_idx(k, st):
            sbuf, dbuf, vv, _, _, _, isem, _ = st
            s = chunk_of(k)
            pltpu.async_copy(ei_hbm.at[0].at[s], sbuf, isem)
            pltpu.async_copy(ei_hbm.at[1].at[s], dbuf, isem)
            pltpu.async_copy(vals_hbm.at[s], vv, isem)

        def wait_idx(k, st):
            sbuf, dbuf, vv, _, _, _, isem, _ = st
            s = chunk_of(k)
            pltpu.make_async_copy(ei_hbm.at[0].at[s], sbuf, isem).wait()
            pltpu.make_async_copy(ei_hbm.at[1].at[s], dbuf, isem).wait()
            pltpu.make_async_copy(vals_hbm.at[s], vv, isem).wait()

        def issue_gather(st):
            sbuf, _, _, _, rows, gsem, _, _ = st
            pltpu.async_copy(x_hbm.at[sbuf], rows, gsem)

        def wait_gather(st):
            sbuf, _, _, _, rows, gsem, _, _ = st
            pltpu.make_async_copy(x_hbm.at[sbuf], rows, gsem).wait()

        def compute(st):
            _, _, vv, _, rows, _, _, _ = st

            def gbody(g, carry):
                vg = vv[g, :]
                for j in range(L):
                    vj = _lane_splat(vg, j)
                    gi = g * L + j
                    for c in range(d // L):
                        sl = slice(c * L, (c + 1) * L)
                        rows[gi, sl] = rows[gi, sl] * vj
                return carry
            lax.fori_loop(0, K // L, gbody, 0)

        def issue_scatter(st):
            _, dbuf, _, didx, rows, _, _, ssem = st
            for c in range(K // L):
                didx[c * L:(c + 1) * L] = dbuf[c * L:(c + 1) * L]
            pltpu.async_copy(rows, acc_sh.at[didx], ssem, add=True)

        def wait_scatter(st):
            _, _, _, didx, rows, _, _, ssem = st
            pltpu.make_async_copy(rows, acc_sh.at[didx], ssem).wait()

        issue_idx(0, sets[0])
        issue_idx(1, sets[1])

        for r in range(8):
            for c in range(d // L):
                zrow_v[r, c * L:(c + 1) * L] = jnp.zeros((L,), jnp.float32)

        def init_issue(i, carry):
            rchunk = sid + i * NS

            @pl.when(rchunk < nrc)
            def _():
                @pl.when(rchunk % NC == cid)
                def _():
                    pltpu.async_copy(
                        x_hbm.at[pl.ds(rchunk * RC, RC)],
                        acc_sh.at[pl.ds(rchunk * RC, RC)], zsem)

                @pl.when(rchunk % NC != cid)
                def _():
                    for jj in range(RC // 8):
                        pltpu.async_copy(
                            zrow_v, acc_sh.at[pl.ds(rchunk * RC + jj * 8, 8)],
                            zsem)
            return carry
        lax.fori_loop(0, (nrc + NS - 1) // NS, init_issue, 0)

        wait_idx(0, sets[0])
        issue_gather(sets[0])

        def init_wait(i, carry):
            rchunk = sid + i * NS

            @pl.when(rchunk < nrc)
            def _():
                @pl.when(rchunk % NC == cid)
                def _():
                    pltpu.make_async_copy(
                        x_hbm.at[pl.ds(rchunk * RC, RC)],
                        acc_sh.at[pl.ds(rchunk * RC, RC)], zsem).wait()

                @pl.when(rchunk % NC != cid)
                def _():
                    for jj in range(RC // 8):
                        pltpu.make_async_copy(
                            zrow_v, acc_sh.at[pl.ds(rchunk * RC + jj * 8, 8)],
                            zsem).wait()
            return carry
        lax.fori_loop(0, (nrc + NS - 1) // NS, init_wait, 0)
        plsc.subcore_barrier()

        has_extra = wid < extra

        def stage(k, cur, nxt, guard_next, guard_nn, first=False):
            @pl.when(guard_next)
            def _():
                wait_idx(k + 1, nxt)
                if not first:
                    wait_scatter(nxt)
                issue_gather(nxt)
            wait_gather(cur)
            compute(cur)
            issue_scatter(cur)

            @pl.when(guard_nn)
            def _():
                issue_idx(k + 2, cur)

        def pair_body(i, carry):
            k0 = 2 * i
            last = i == npair - 1
            stage(k0, sets[0], sets[1],
                  jnp.bool_(True),
                  jnp.logical_or(~last, has_extra))
            stage(k0 + 1, sets[1], sets[0],
                  jnp.logical_or(~last, has_extra),
                  ~last)
            return carry

        stage(0, sets[0], sets[1], jnp.bool_(True),
              jnp.bool_(npair > 1) | has_extra, first=True)
        stage(1, sets[1], sets[0],
              jnp.bool_(npair > 1) | has_extra,
              jnp.bool_(npair > 1))
        lax.fori_loop(1, npair, pair_body, 0)

        @pl.when(has_extra)
        def _():
            wait_gather(sets[0])
            compute(sets[0])
            issue_scatter(sets[0])
        wait_scatter(sets[1])
        wait_scatter(sets[0])

        plsc.subcore_barrier()

        def rb_issue(i, carry):
            rchunk = sid + i * NS

            @pl.when(rchunk < nrc)
            def _():
                r = rchunk * RC
                pltpu.async_copy(acc_sh.at[pl.ds(r, RC)],
                                 out_hbm.at[cid, pl.ds(r, RC)], zsem)
            return carry
        lax.fori_loop(0, (nrc + NS - 1) // NS, rb_issue, 0)

        def rb_wait(i, carry):
            rchunk = sid + i * NS

            @pl.when(rchunk < nrc)
            def _():
                r = rchunk * RC
                pltpu.make_async_copy(acc_sh.at[pl.ds(r, RC)],
                                      out_hbm.at[cid, pl.ds(r, RC)],
                                      zsem).wait()
            return carry
        lax.fori_loop(0, (nrc + NS - 1) // NS, rb_wait, 0)

    return sc_kernel(x, ei3, vals3d)


def _dense_body(acc_ref, w_ref, b_ref, o_ref):
    a = acc_ref[0] + acc_ref[1]
    s = jnp.dot(a, w_ref[...], preferred_element_type=jnp.float32) + b_ref[...]
    alpha = 1.6732632423543772
    scale = 1.0507009873554805
    o_ref[...] = scale * jnp.where(s > 0, s, alpha * (jnp.exp(s) - 1.0))


def _tc_dense(acc, w, b):
    nc, n, d = acc.shape
    h = w.shape[1]
    bm = 2000
    grid = (n // bm,)
    return pl.pallas_call(
        _dense_body,
        grid=grid,
        in_specs=[
            pl.BlockSpec((NC, bm, d), lambda i: (0, i, 0)),
            pl.BlockSpec((d, h), lambda i: (0, 0)),
            pl.BlockSpec((1, h), lambda i: (0, 0)),
        ],
        out_specs=pl.BlockSpec((bm, h), lambda i: (i, 0)),
        out_shape=jax.ShapeDtypeStruct((n, h), jnp.float32),
    )(acc, w, b)


@jax.jit
def kernel(x, edge_index, edge_vals, W0, b0):
    e = edge_vals.shape[0]
    nsub = e // K
    ei3 = edge_index.astype(jnp.int32).reshape(2, nsub, K)
    vals3d = edge_vals.astype(jnp.float32).reshape(nsub, K // L, L)
    acc = _sc_edge_kernel(x, ei3, vals3d)
    return _tc_dense(acc, W0, b0.reshape(1, -1))
